# baseline (speedup 1.0000x reference)
import functools

import jax
import jax.numpy as jnp
from jax import lax
from jax.experimental import pallas as pl
from jax.experimental.pallas import tpu as pltpu
from jax.experimental.pallas import tpu_sc as plsc

N = 10000
E = 160000
IN = 128
H = 8
C = 64
HC = H * C
L = 4
NT = 50
NF = 100
TE = 16
B = 64
SKIP = [False, True, True, True]

NP = 10240
CH = 32
NCH = 313
NW = 32
SEG = 2048
ET = E + N
EPAD = ET + SEG + 16
NOFF = NP + 16

@functools.lru_cache(maxsize=1)
def _mesh():
    return plsc.VectorSubcoreMesh(core_axis_name="c", subcore_axis_name="s")


def _bf(a):
    return a.astype(jnp.bfloat16)


def _proj_body(x_ref, wl_ref, wr_ref, xl_ref, xr_ref):
    x = _bf(x_ref[...])
    xl_ref[...] = jnp.dot(x, _bf(wl_ref[...]), preferred_element_type=jnp.float32)
    xr_ref[...] = jnp.dot(x, _bf(wr_ref[...]), preferred_element_type=jnp.float32)


def _proj(x):
    d = x.shape[1]
    return pl.pallas_call(
        _proj_body,
        grid=(NP // 256,),
        in_specs=[
            pl.BlockSpec((256, d), lambda i: (i, 0)),
            pl.BlockSpec((d, HC), lambda i: (0, 0)),
            pl.BlockSpec((d, HC), lambda i: (0, 0)),
        ],
        out_specs=[
            pl.BlockSpec((256, HC), lambda i: (i, 0)),
            pl.BlockSpec((256, HC), lambda i: (i, 0)),
        ],
        out_shape=[
            jax.ShapeDtypeStruct((NP, HC), jnp.float32),
            jax.ShapeDtypeStruct((NP, HC), jnp.float32),
        ],
    )


def _mid_body(gat_ref, h_ref, b_ref, g_ref, be_ref, wl_ref, wr_ref,
              hn_ref, xl_ref, xr_ref, *, skip):
    z = jnp.maximum(gat_ref[...] + b_ref[...], 0.0)
    mu = jnp.mean(z, axis=1, keepdims=True)
    var = jnp.mean((z - mu) ** 2, axis=1, keepdims=True)
    z = (z - mu) * lax.rsqrt(var + 1e-5) * g_ref[...] + be_ref[...]
    h = z + h_ref[...] if skip else z
    hn_ref[...] = h
    hb = _bf(h)
    xl_ref[...] = jnp.dot(hb, _bf(wl_ref[...]), preferred_element_type=jnp.float32)
    xr_ref[...] = jnp.dot(hb, _bf(wr_ref[...]), preferred_element_type=jnp.float32)


def _mid(gat, h, b2, g2, be2, wl, wr, skip):
    return pl.pallas_call(
        functools.partial(_mid_body, skip=skip),
        grid=(NP // 256,),
        in_specs=[
            pl.BlockSpec((256, HC), lambda i: (i, 0)),
            pl.BlockSpec((256, HC), lambda i: (i, 0)),
            pl.BlockSpec((1, HC), lambda i: (0, 0)),
            pl.BlockSpec((1, HC), lambda i: (0, 0)),
            pl.BlockSpec((1, HC), lambda i: (0, 0)),
            pl.BlockSpec((HC, HC), lambda i: (0, 0)),
            pl.BlockSpec((HC, HC), lambda i: (0, 0)),
        ],
        out_specs=[
            pl.BlockSpec((256, HC), lambda i: (i, 0)),
            pl.BlockSpec((256, HC), lambda i: (i, 0)),
            pl.BlockSpec((256, HC), lambda i: (i, 0)),
        ],
        out_shape=[
            jax.ShapeDtypeStruct((NP, HC), jnp.float32),
            jax.ShapeDtypeStruct((NP, HC), jnp.float32),
            jax.ShapeDtypeStruct((NP, HC), jnp.float32),
        ],
    )(gat, h, b2, g2, be2, wl, wr)


def _final_body(gat_ref, h_ref, b_ref, g_ref, be_ref, bat_ref, oh_ref,
                embp_ref, wfam_ref, bfam_ref, wtyp_ref, btyp_ref,
                s_ref, mx_ref, cnt_ref, o1_ref, o2_ref):
    i = pl.program_id(0)
    z = jnp.maximum(gat_ref[...] + b_ref[...], 0.0)
    mu = jnp.mean(z, axis=1, keepdims=True)
    var = jnp.mean((z - mu) ** 2, axis=1, keepdims=True)
    z = (z - mu) * lax.rsqrt(var + 1e-5) * g_ref[...] + be_ref[...]
    h = z + h_ref[...]
    bb = bat_ref[...]
    h = jnp.where(bb == B, 0.0, h)

    @pl.when(i == 0)
    def _():
        s_ref[...] = jnp.zeros((B, HC), jnp.float32)
        mx_ref[...] = jnp.full((B, HC), -1e30, jnp.float32)
        cnt_ref[...] = jnp.zeros((B, 128), jnp.float32)

    onehot = (bb == lax.broadcasted_iota(jnp.int32, (1, B), 1)).astype(jnp.float32)
    s_ref[...] += lax.dot_general(onehot, h, (((0,), (0,)), ((), ())),
                                  preferred_element_type=jnp.float32)
    cnt_ref[...] += jnp.broadcast_to(
        jnp.sum(onehot, axis=0)[:, None], (B, 128))
    hm = jnp.where(bb >= 0, h, -1e30)
    for bidx in range(B):
        sel = jnp.where(bb == bidx, hm, -1e30)
        mx_ref[pl.ds(bidx, 1), :] = jnp.maximum(
            mx_ref[pl.ds(bidx, 1), :], jnp.max(sel, axis=0, keepdims=True))

    @pl.when(i == NP // 256 - 1)
    def _():
        s = s_ref[...]
        cnt = jnp.maximum(cnt_ref[:, 0:1], 1.0)
        m1 = s / cnt
        mx = mx_ref[...]
        mx = jnp.where(mx < -1e29, 0.0, mx)
        embg = jnp.dot(oh_ref[...], embp_ref[...],
                       preferred_element_type=jnp.float32)
        wfam = wfam_ref[...]
        wtyp = wtyp_ref[...]

        def head(w, bias):
            r = jnp.dot(m1, w[0:HC], preferred_element_type=jnp.float32)
            r += jnp.dot(mx, w[HC:2 * HC], preferred_element_type=jnp.float32)
            r += jnp.dot(s, w[2 * HC:3 * HC], preferred_element_type=jnp.float32)
            r += jnp.dot(embg, w[3 * HC:3 * HC + 128],
                         preferred_element_type=jnp.float32)
            return r + bias

        o1_ref[...] = head(wfam, bfam_ref[...])
        o2_ref[...] = head(wtyp, btyp_ref[...])


def _final(gat, h, b2, g2, be2, batp, oh, embp, wfamp, bfamp, wtypp, btypp):
    nb = NP // 256
    return pl.pallas_call(
        _final_body,
        grid=(nb,),
        in_specs=[
            pl.BlockSpec((256, HC), lambda i: (i, 0)),
            pl.BlockSpec((256, HC), lambda i: (i, 0)),
            pl.BlockSpec((1, HC), lambda i: (0, 0)),
            pl.BlockSpec((1, HC), lambda i: (0, 0)),
            pl.BlockSpec((1, HC), lambda i: (0, 0)),
            pl.BlockSpec((256, 1), lambda i: (i, 0)),
            pl.BlockSpec((B, B), lambda i: (0, 0)),
            pl.BlockSpec((B, 128), lambda i: (0, 0)),
            pl.BlockSpec((3 * HC + 128, 128), lambda i: (0, 0)),
            pl.BlockSpec((1, 128), lambda i: (0, 0)),
            pl.BlockSpec((3 * HC + 128, 128), lambda i: (0, 0)),
            pl.BlockSpec((1, 128), lambda i: (0, 0)),
        ],
        out_specs=[
            pl.BlockSpec((B, HC), lambda i: (0, 0)),
            pl.BlockSpec((B, HC), lambda i: (0, 0)),
            pl.BlockSpec((B, 128), lambda i: (0, 0)),
            pl.BlockSpec((B, 128), lambda i: (0, 0)),
            pl.BlockSpec((B, 128), lambda i: (0, 0)),
        ],
        out_shape=[
            jax.ShapeDtypeStruct((B, HC), jnp.float32),
            jax.ShapeDtypeStruct((B, HC), jnp.float32),
            jax.ShapeDtypeStruct((B, 128), jnp.float32),
            jax.ShapeDtypeStruct((B, 128), jnp.float32),
            jax.ShapeDtypeStruct((B, 128), jnp.float32),
        ],
    )(gat, h, b2, g2, be2, batp, oh, embp, wfamp, bfamp, wtypp, btypp)


def _rotv(k):
    return (lax.iota(jnp.int32, 16) + k) % 16


def _lanesum(v):
    v = v + jnp.take(v, _rotv(8))
    v = v + jnp.take(v, _rotv(4))
    v = v + jnp.take(v, _rotv(2))
    v = v + jnp.take(v, _rotv(1))
    return v


def _treemin(v):
    v = jnp.minimum(v, jnp.take(v, _rotv(8)))
    v = jnp.minimum(v, jnp.take(v, _rotv(4)))
    v = jnp.minimum(v, jnp.take(v, _rotv(2)))
    v = jnp.minimum(v, jnp.take(v, _rotv(1)))
    return v


def _treemax(v):
    v = jnp.maximum(v, jnp.take(v, _rotv(8)))
    v = jnp.maximum(v, jnp.take(v, _rotv(4)))
    v = jnp.maximum(v, jnp.take(v, _rotv(2)))
    v = jnp.maximum(v, jnp.take(v, _rotv(1)))
    return v


def _edge_work4(u, dl, grows, xr_buf, att_buf, mden, acc):
    iota = lax.iota(jnp.int32, 16)
    rows = [grows.at[u + t] for t in range(4)]
    ls = [jnp.full((16,), -1e30, jnp.float32) for _ in range(4)]
    for h in range(H):
        ps = [jnp.zeros((16,), jnp.float32) for _ in range(4)]
        for q in range(4):
            j = h * 4 + q
            bq = xr_buf[pl.ds(dl * HC + j * 16, 16)]
            aj = att_buf[pl.ds(j * 16, 16)]
            for t in range(4):
                zt = rows[t][pl.ds(j * 16, 16)] + bq
                zt = jnp.maximum(zt, 0.2 * zt)
                ps[t] = ps[t] + zt * aj
        for t in range(4):
            ls[t] = jnp.where(iota == h, _lanesum(ps[t]), ls[t])
    mrow = mden[pl.ds(dl * 32, 16)]
    mnew = jnp.maximum(jnp.maximum(mrow, jnp.maximum(ls[0], ls[1])),
                       jnp.maximum(ls[2], ls[3]))
    r8 = jnp.exp(mrow - mnew)
    es = [jnp.exp(lt - mnew) for lt in ls]
    mden[pl.ds(dl * 32, 16)] = mnew
    mden[pl.ds(dl * 32 + 16, 16)] = (
        mden[pl.ds(dl * 32 + 16, 16)] * r8 + ((es[0] + es[1])
                                              + (es[2] + es[3])))
    for h in range(H):
        sh = r8[h]
        ehs = [es[t][h] for t in range(4)]
        for q in range(4):
            j = h * 4 + q
            o = dl * HC + j * 16
            v = acc[pl.ds(o, 16)] * sh
            for t in range(4):
                v = v + ehs[t] * rows[t][pl.ds(j * 16, 16)]
            acc[pl.ds(o, 16)] = v


def _edge_work2(u, dl, grows, xr_buf, att_buf, mden, acc):
    iota = lax.iota(jnp.int32, 16)
    row1 = grows.at[u]
    row2 = grows.at[u + 1]
    l1 = jnp.full((16,), -1e30, jnp.float32)
    l2 = jnp.full((16,), -1e30, jnp.float32)
    for h in range(H):
        p1 = jnp.zeros((16,), jnp.float32)
        p2 = jnp.zeros((16,), jnp.float32)
        for q in range(4):
            j = h * 4 + q
            bq = xr_buf[pl.ds(dl * HC + j * 16, 16)]
            aj = att_buf[pl.ds(j * 16, 16)]
            a1 = row1[pl.ds(j * 16, 16)]
            a2 = row2[pl.ds(j * 16, 16)]
            z1 = a1 + bq
            z1 = jnp.maximum(z1, 0.2 * z1)
            z2 = a2 + bq
            z2 = jnp.maximum(z2, 0.2 * z2)
            p1 = p1 + z1 * aj
            p2 = p2 + z2 * aj
        l1 = jnp.where(iota == h, _lanesum(p1), l1)
        l2 = jnp.where(iota == h, _lanesum(p2), l2)
    mrow = mden[pl.ds(dl * 32, 16)]
    mnew = jnp.maximum(mrow, jnp.maximum(l1, l2))
    r8 = jnp.exp(mrow - mnew)
    e1 = jnp.exp(l1 - mnew)
    e2 = jnp.exp(l2 - mnew)
    mden[pl.ds(dl * 32, 16)] = mnew
    mden[pl.ds(dl * 32 + 16, 16)] = (
        mden[pl.ds(dl * 32 + 16, 16)] * r8 + e1 + e2)
    for h in range(H):
        sh = r8[h]
        eh1 = e1[h]
        eh2 = e2[h]
        for q in range(4):
            j = h * 4 + q
            o = dl * HC + j * 16
            acc[pl.ds(o, 16)] = (acc[pl.ds(o, 16)] * sh
                                 + eh1 * row1[pl.ds(j * 16, 16)]
                                 + eh2 * row2[pl.ds(j * 16, 16)])


def _edge_work(u, dl, grows, xr_buf, att_buf, mden, acc):
    iota = lax.iota(jnp.int32, 16)
    row = grows.at[u]
    lvec = jnp.full((16,), -1e30, jnp.float32)
    for h in range(H):
        ph = jnp.zeros((16,), jnp.float32)
        for q in range(4):
            j = h * 4 + q
            a = row[pl.ds(j * 16, 16)]
            bq = xr_buf[pl.ds(dl * HC + j * 16, 16)]
            z = a + bq
            z = jnp.maximum(z, 0.2 * z)
            ph = ph + z * att_buf[pl.ds(j * 16, 16)]
        lvec = jnp.where(iota == h, _lanesum(ph), lvec)
    mrow = mden[pl.ds(dl * 32, 16)]
    mnew = jnp.maximum(mrow, lvec)
    r8 = jnp.exp(mrow - mnew)
    e8 = jnp.exp(lvec - mnew)
    mden[pl.ds(dl * 32, 16)] = mnew
    mden[pl.ds(dl * 32 + 16, 16)] = mden[pl.ds(dl * 32 + 16, 16)] * r8 + e8
    for h in range(H):
        sh = r8[h]
        eh = e8[h]
        for q in range(4):
            j = h * 4 + q
            o = dl * HC + j * 16
            acc[pl.ds(o, 16)] = acc[pl.ds(o, 16)] * sh + eh * row[pl.ds(j * 16, 16)]


@functools.lru_cache(maxsize=1)
def _sc_edge_kernel():
    return functools.partial(
        pl.kernel, mesh=_mesh(),
        out_type=jax.ShapeDtypeStruct((NP * HC,), jnp.float32),
        scratch_types=[
            pltpu.VMEM((SEG,), jnp.int32),
            pltpu.VMEM((CH * HC,), jnp.float32),
            pltpu.VMEM((CH * HC,), jnp.float32),
            pltpu.VMEM((CH * 32,), jnp.float32),
            pltpu.VMEM((2, 16, HC), jnp.float32),
            pltpu.VMEM((512,), jnp.float32),
            pltpu.VMEM((NOFF,), jnp.int32),
            pltpu.SemaphoreType.DMA((2,)),
        ],
    )(_sc_edge_body)


def _sc_edge_body(xl_hbm, xr_hbm, pk_hbm, noff_hbm, att_hbm, out_hbm,
                  ebuf, xr_buf, acc, mden, grows2, att_buf, noff_buf, sems):
    w = lax.axis_index("s") * 2 + lax.axis_index("c")
    pltpu.sync_copy(noff_hbm, noff_buf)
    pltpu.sync_copy(att_hbm, att_buf)
    iota = lax.iota(jnp.int32, 16)
    zf = jnp.zeros((16,), jnp.float32)
    neg = jnp.full((16,), -1e30, jnp.float32)

    def chunk_body(k, carry):
        c = w + k * NW

        @pl.when(c < NCH)
        def _():
            base = c * CH
            bo = pl.multiple_of(base * HC, 16384)
            pltpu.sync_copy(xr_hbm.at[pl.ds(bo, CH * HC)],
                            xr_buf.at[pl.ds(0, CH * HC)])

            def zacc(j, cc):
                acc[pl.ds(j * 16, 16)] = zf
                return cc

            lax.fori_loop(0, CH * HC // 16, zacc, 0)

            def ztab(j, cc):
                mden[pl.ds(j * 32, 16)] = neg
                mden[pl.ds(j * 32 + 16, 16)] = zf
                return cc

            lax.fori_loop(0, CH, ztab, 0)

            start_c = noff_buf[pl.ds(base, 16)][0]
            end_c = noff_buf[pl.ds(base + CH, 16)][0]
            start_al = start_c & -8
            nseg = (end_c - start_al + SEG - 1) // SEG

            def seg_body(sg, cc):
                seg_base = pl.multiple_of(start_al + sg * SEG, 8)
                pltpu.sync_copy(pk_hbm.at[pl.ds(seg_base, SEG)], ebuf)
                ng = jnp.minimum(SEG // 16, (end_c - seg_base + 15) // 16)

                def srcg_of(g):
                    pkv = ebuf[pl.ds(g * 16, 16)]
                    pos = seg_base + g * 16 + iota
                    val = jnp.logical_and(pos >= start_c, pos < end_c)
                    return pkv, val, jnp.where(
                        val, lax.bitwise_and(pkv, 16383), 0)

                def issue(g):
                    _, _, srcg = srcg_of(g)
                    pltpu.async_copy(xl_hbm.at[srcg], grows2.at[g & 1],
                                     sems.at[g & 1])

                issue(0)

                def grp_body(g, cc2):
                    @pl.when(g + 1 < ng)
                    def _():
                        issue(g + 1)
                    pkv, val, srcg = srcg_of(g)
                    pltpu.make_async_copy(
                        xl_hbm.at[srcg], grows2.at[g & 1],
                        sems.at[g & 1]).wait()
                    pos0 = seg_base + g * 16
                    grows = grows2.at[g & 1]
                    dlv = lax.shift_right_logical(pkv, 14) - base
                    dl_lo = _treemin(jnp.where(val, dlv, CH - 1))[0]
                    dl_hi = _treemax(jnp.where(val, dlv, 0))[0]

                    def run_body(dl, cc3):
                        ovn = noff_buf[pl.ds(base + dl, 16)]
                        ulo = jnp.maximum(ovn[0] - pos0, 0)
                        uhi = jnp.minimum(ovn[1] - pos0, 16)
                        nn = jnp.maximum(uhi - ulo, 0)

                        def quad_body(t, cc4):
                            _edge_work4(ulo + t * 4, dl, grows, xr_buf,
                                        att_buf, mden, acc)
                            return cc4

                        lax.fori_loop(0, nn // 4, quad_body, 0)
                        rem = nn % 4

                        @pl.when(rem >= 2)
                        def _():
                            def pair_tail(t, cc4):
                                _edge_work2(uhi - rem, dl, grows, xr_buf,
                                            att_buf, mden, acc)
                                return cc4

                            lax.fori_loop(0, 1, pair_tail, 0)

                        @pl.when(rem % 2 == 1)
                        def _():
                            def tail_body(u, cc4):
                                _edge_work(u, dl, grows, xr_buf, att_buf,
                                           mden, acc)
                                return cc4

                            lax.fori_loop(uhi - 1, uhi, tail_body, 0)

                        return cc3

                    lax.fori_loop(dl_lo, dl_hi + 1, run_body, 0)
                    return cc2

                lax.fori_loop(0, ng, grp_body, 0)
                return cc

            lax.fori_loop(0, nseg, seg_body, 0)

            def fin_body(dl, cc):
                dv = mden[pl.ds(dl * 32 + 16, 16)]
                inv = 1.0 / (dv + 1e-16)
                for h in range(H):
                    ih = inv[h]
                    for q in range(4):
                        o = dl * HC + (h * 4 + q) * 16
                        acc[pl.ds(o, 16)] = acc[pl.ds(o, 16)] * ih
                return cc

            lax.fori_loop(0, CH, fin_body, 0)
            pltpu.sync_copy(acc.at[pl.ds(0, CH * HC)],
                            out_hbm.at[pl.ds(bo, CH * HC)])

        return carry

    lax.fori_loop(0, (NCH + NW - 1) // NW, chunk_body, 0)


def _pad_rows(a, rows):
    return jnp.pad(a, ((0, rows - a.shape[0]),) + ((0, 0),) * (a.ndim - 1))


def kernel(x, edge, batch, y_type, Wl0, Wr0, att0, b0, g0, be0,
           Wl1, Wr1, att1, b1, g1, be1, Wl2, Wr2, att2, b2, g2, be2,
           Wl3, Wr3, att3, b3, g3, be3, emb, Wfam, bfam, Wtyp, btyp):
    Wls = [Wl0, Wl1, Wl2, Wl3]
    Wrs = [Wr0, Wr1, Wr2, Wr3]
    atts = [att0, att1, att2, att3]
    bs = [b0, b1, b2, b3]
    gs = [g0, g1, g2, g3]
    bes = [be0, be1, be2, be3]

    loop = jnp.arange(N, dtype=jnp.int32)
    src = jnp.concatenate([edge[0].astype(jnp.int32), loop])
    dst = jnp.concatenate([edge[1].astype(jnp.int32), loop])
    pk = jnp.sort(dst * 16384 + src)
    noff = jnp.searchsorted(pk, jnp.arange(NOFF, dtype=jnp.int32) * 16384,
                            side='left').astype(jnp.int32)
    pk = jnp.pad(pk, (0, EPAD - ET), constant_values=16383 * 16384)

    xp = _pad_rows(x, NP)
    batp = jnp.pad(batch.astype(jnp.int32), (0, NP - N),
                   constant_values=B).reshape(NP, 1)
    oh = (y_type[:, None] == jnp.arange(B)[None, :]).astype(jnp.float32)
    embp = jnp.pad(emb, ((0, B - NT), (0, 128 - TE)))
    wf4 = jnp.pad(Wfam[3 * HC:], ((0, 128 - TE), (0, 0)))
    wfamp = jnp.pad(jnp.concatenate([Wfam[:3 * HC], wf4], axis=0),
                    ((0, 0), (0, 128 - NF)))
    wt4 = jnp.pad(Wtyp[3 * HC:], ((0, 128 - TE), (0, 0)))
    wtypp = jnp.pad(jnp.concatenate([Wtyp[:3 * HC], wt4], axis=0),
                    ((0, 0), (0, 128 - NT)))
    bfamp = jnp.pad(bfam, (0, 128 - NF)).reshape(1, 128)
    btypp = jnp.pad(btyp, (0, 128 - NT)).reshape(1, 128)

    h = xp
    xl, xr = _proj(xp)(xp, Wls[0], Wrs[0])
    for i in range(L):
        gat_flat = _sc_edge_kernel()(xl, xr.reshape(-1), pk, noff,
                                     atts[i].reshape(-1))
        gat = gat_flat.reshape(NP, HC)
        b2 = bs[i].reshape(1, HC)
        g2 = gs[i].reshape(1, HC)
        be2 = bes[i].reshape(1, HC)
        if i < L - 1:
            h, xl, xr = _mid(gat, h, b2, g2, be2, Wls[i + 1], Wrs[i + 1],
                             SKIP[i])
        else:
            _, _, _, o1, o2 = _final(gat, h, b2, g2, be2, batp, oh, embp,
                                     wfamp, bfamp, wtypp, btypp)
    return (o1[:, :NF], o2[:, :NT])


# speedup vs baseline: 1.0008x; 1.0008x over previous
import functools

import jax
import jax.numpy as jnp
from jax import lax
from jax.experimental import pallas as pl
from jax.experimental.pallas import tpu as pltpu
from jax.experimental.pallas import tpu_sc as plsc

N = 10000
E = 160000
IN = 128
H = 8
C = 64
HC = H * C
L = 4
NT = 50
NF = 100
TE = 16
B = 64
SKIP = [False, True, True, True]

NP = 10240
CH = 32
NCH = 313
NW = 32
SEG = 2048
ET = E + N
EPAD = ET + SEG + 16
NOFF = NP + 16

@functools.lru_cache(maxsize=1)
def _mesh():
    return plsc.VectorSubcoreMesh(core_axis_name="c", subcore_axis_name="s")


def _proj_body(x_ref, wl_ref, wr_ref, xl_ref, xr_ref):
    x = x_ref[...]
    xl_ref[...] = jnp.dot(x, wl_ref[...], preferred_element_type=jnp.float32)
    xr_ref[...] = jnp.dot(x, wr_ref[...], preferred_element_type=jnp.float32)


def _proj(x):
    d = x.shape[1]
    return pl.pallas_call(
        _proj_body,
        grid=(NP // 256,),
        in_specs=[
            pl.BlockSpec((256, d), lambda i: (i, 0)),
            pl.BlockSpec((d, HC), lambda i: (0, 0)),
            pl.BlockSpec((d, HC), lambda i: (0, 0)),
        ],
        out_specs=[
            pl.BlockSpec((256, HC), lambda i: (i, 0)),
            pl.BlockSpec((256, HC), lambda i: (i, 0)),
        ],
        out_shape=[
            jax.ShapeDtypeStruct((NP, HC), jnp.float32),
            jax.ShapeDtypeStruct((NP, HC), jnp.float32),
        ],
    )


def _mid_body(gat_ref, h_ref, b_ref, g_ref, be_ref, wl_ref, wr_ref,
              hn_ref, xl_ref, xr_ref, *, skip):
    z = jnp.maximum(gat_ref[...] + b_ref[...], 0.0)
    mu = jnp.mean(z, axis=1, keepdims=True)
    var = jnp.mean((z - mu) ** 2, axis=1, keepdims=True)
    z = (z - mu) * lax.rsqrt(var + 1e-5) * g_ref[...] + be_ref[...]
    h = z + h_ref[...] if skip else z
    hn_ref[...] = h
    xl_ref[...] = jnp.dot(h, wl_ref[...], preferred_element_type=jnp.float32)
    xr_ref[...] = jnp.dot(h, wr_ref[...], preferred_element_type=jnp.float32)


def _mid(gat, h, b2, g2, be2, wl, wr, skip):
    return pl.pallas_call(
        functools.partial(_mid_body, skip=skip),
        grid=(NP // 256,),
        in_specs=[
            pl.BlockSpec((256, HC), lambda i: (i, 0)),
            pl.BlockSpec((256, HC), lambda i: (i, 0)),
            pl.BlockSpec((1, HC), lambda i: (0, 0)),
            pl.BlockSpec((1, HC), lambda i: (0, 0)),
            pl.BlockSpec((1, HC), lambda i: (0, 0)),
            pl.BlockSpec((HC, HC), lambda i: (0, 0)),
            pl.BlockSpec((HC, HC), lambda i: (0, 0)),
        ],
        out_specs=[
            pl.BlockSpec((256, HC), lambda i: (i, 0)),
            pl.BlockSpec((256, HC), lambda i: (i, 0)),
            pl.BlockSpec((256, HC), lambda i: (i, 0)),
        ],
        out_shape=[
            jax.ShapeDtypeStruct((NP, HC), jnp.float32),
            jax.ShapeDtypeStruct((NP, HC), jnp.float32),
            jax.ShapeDtypeStruct((NP, HC), jnp.float32),
        ],
    )(gat, h, b2, g2, be2, wl, wr)


def _final_body(gat_ref, h_ref, b_ref, g_ref, be_ref, bat_ref, oh_ref,
                embp_ref, wfam_ref, bfam_ref, wtyp_ref, btyp_ref,
                s_ref, mx_ref, cnt_ref, o1_ref, o2_ref):
    i = pl.program_id(0)
    z = jnp.maximum(gat_ref[...] + b_ref[...], 0.0)
    mu = jnp.mean(z, axis=1, keepdims=True)
    var = jnp.mean((z - mu) ** 2, axis=1, keepdims=True)
    z = (z - mu) * lax.rsqrt(var + 1e-5) * g_ref[...] + be_ref[...]
    h = z + h_ref[...]
    bb = bat_ref[...]
    h = jnp.where(bb == B, 0.0, h)

    @pl.when(i == 0)
    def _():
        s_ref[...] = jnp.zeros((B, HC), jnp.float32)
        mx_ref[...] = jnp.full((B, HC), -1e30, jnp.float32)
        cnt_ref[...] = jnp.zeros((B, 128), jnp.float32)

    onehot = (bb == lax.broadcasted_iota(jnp.int32, (1, B), 1)).astype(jnp.float32)
    s_ref[...] += lax.dot_general(onehot, h, (((0,), (0,)), ((), ())),
                                  preferred_element_type=jnp.float32)
    cnt_ref[...] += jnp.broadcast_to(
        jnp.sum(onehot, axis=0)[:, None], (B, 128))
    hm = jnp.where(bb >= 0, h, -1e30)
    for bidx in range(B):
        sel = jnp.where(bb == bidx, hm, -1e30)
        mx_ref[pl.ds(bidx, 1), :] = jnp.maximum(
            mx_ref[pl.ds(bidx, 1), :], jnp.max(sel, axis=0, keepdims=True))

    @pl.when(i == NP // 256 - 1)
    def _():
        s = s_ref[...]
        cnt = jnp.maximum(cnt_ref[:, 0:1], 1.0)
        m1 = s / cnt
        mx = mx_ref[...]
        mx = jnp.where(mx < -1e29, 0.0, mx)
        embg = jnp.dot(oh_ref[...], embp_ref[...],
                       preferred_element_type=jnp.float32)
        wfam = wfam_ref[...]
        wtyp = wtyp_ref[...]

        def head(w, bias):
            r = jnp.dot(m1, w[0:HC], preferred_element_type=jnp.float32)
            r += jnp.dot(mx, w[HC:2 * HC], preferred_element_type=jnp.float32)
            r += jnp.dot(s, w[2 * HC:3 * HC], preferred_element_type=jnp.float32)
            r += jnp.dot(embg, w[3 * HC:3 * HC + 128],
                         preferred_element_type=jnp.float32)
            return r + bias

        o1_ref[...] = head(wfam, bfam_ref[...])
        o2_ref[...] = head(wtyp, btyp_ref[...])


def _final(gat, h, b2, g2, be2, batp, oh, embp, wfamp, bfamp, wtypp, btypp):
    nb = NP // 256
    return pl.pallas_call(
        _final_body,
        grid=(nb,),
        in_specs=[
            pl.BlockSpec((256, HC), lambda i: (i, 0)),
            pl.BlockSpec((256, HC), lambda i: (i, 0)),
            pl.BlockSpec((1, HC), lambda i: (0, 0)),
            pl.BlockSpec((1, HC), lambda i: (0, 0)),
            pl.BlockSpec((1, HC), lambda i: (0, 0)),
            pl.BlockSpec((256, 1), lambda i: (i, 0)),
            pl.BlockSpec((B, B), lambda i: (0, 0)),
            pl.BlockSpec((B, 128), lambda i: (0, 0)),
            pl.BlockSpec((3 * HC + 128, 128), lambda i: (0, 0)),
            pl.BlockSpec((1, 128), lambda i: (0, 0)),
            pl.BlockSpec((3 * HC + 128, 128), lambda i: (0, 0)),
            pl.BlockSpec((1, 128), lambda i: (0, 0)),
        ],
        out_specs=[
            pl.BlockSpec((B, HC), lambda i: (0, 0)),
            pl.BlockSpec((B, HC), lambda i: (0, 0)),
            pl.BlockSpec((B, 128), lambda i: (0, 0)),
            pl.BlockSpec((B, 128), lambda i: (0, 0)),
            pl.BlockSpec((B, 128), lambda i: (0, 0)),
        ],
        out_shape=[
            jax.ShapeDtypeStruct((B, HC), jnp.float32),
            jax.ShapeDtypeStruct((B, HC), jnp.float32),
            jax.ShapeDtypeStruct((B, 128), jnp.float32),
            jax.ShapeDtypeStruct((B, 128), jnp.float32),
            jax.ShapeDtypeStruct((B, 128), jnp.float32),
        ],
    )(gat, h, b2, g2, be2, batp, oh, embp, wfamp, bfamp, wtypp, btypp)


def _rotv(k):
    return (lax.iota(jnp.int32, 16) + k) % 16


def _lanesum(v):
    v = v + jnp.take(v, _rotv(8))
    v = v + jnp.take(v, _rotv(4))
    v = v + jnp.take(v, _rotv(2))
    v = v + jnp.take(v, _rotv(1))
    return v


def _treemin(v):
    v = jnp.minimum(v, jnp.take(v, _rotv(8)))
    v = jnp.minimum(v, jnp.take(v, _rotv(4)))
    v = jnp.minimum(v, jnp.take(v, _rotv(2)))
    v = jnp.minimum(v, jnp.take(v, _rotv(1)))
    return v


def _treemax(v):
    v = jnp.maximum(v, jnp.take(v, _rotv(8)))
    v = jnp.maximum(v, jnp.take(v, _rotv(4)))
    v = jnp.maximum(v, jnp.take(v, _rotv(2)))
    v = jnp.maximum(v, jnp.take(v, _rotv(1)))
    return v


def _edge_work4(u, dl, grows, xr_buf, att_buf, mden, acc):
    iota = lax.iota(jnp.int32, 16)
    rows = [grows.at[u + t] for t in range(4)]
    ls = [jnp.full((16,), -1e30, jnp.float32) for _ in range(4)]
    for h in range(H):
        ps = [jnp.zeros((16,), jnp.float32) for _ in range(4)]
        for q in range(4):
            j = h * 4 + q
            bq = xr_buf[pl.ds(dl * HC + j * 16, 16)]
            aj = att_buf[pl.ds(j * 16, 16)]
            for t in range(4):
                zt = rows[t][pl.ds(j * 16, 16)] + bq
                zt = jnp.maximum(zt, 0.2 * zt)
                ps[t] = ps[t] + zt * aj
        for t in range(4):
            ls[t] = jnp.where(iota == h, _lanesum(ps[t]), ls[t])
    mrow = mden[pl.ds(dl * 32, 16)]
    mnew = jnp.maximum(jnp.maximum(mrow, jnp.maximum(ls[0], ls[1])),
                       jnp.maximum(ls[2], ls[3]))
    r8 = jnp.exp(mrow - mnew)
    es = [jnp.exp(lt - mnew) for lt in ls]
    mden[pl.ds(dl * 32, 16)] = mnew
    mden[pl.ds(dl * 32 + 16, 16)] = (
        mden[pl.ds(dl * 32 + 16, 16)] * r8 + ((es[0] + es[1])
                                              + (es[2] + es[3])))
    for h in range(H):
        sh = r8[h]
        ehs = [es[t][h] for t in range(4)]
        for q in range(4):
            j = h * 4 + q
            o = dl * HC + j * 16
            v = acc[pl.ds(o, 16)] * sh
            for t in range(4):
                v = v + ehs[t] * rows[t][pl.ds(j * 16, 16)]
            acc[pl.ds(o, 16)] = v


def _edge_work2(u, dl, grows, xr_buf, att_buf, mden, acc):
    iota = lax.iota(jnp.int32, 16)
    row1 = grows.at[u]
    row2 = grows.at[u + 1]
    l1 = jnp.full((16,), -1e30, jnp.float32)
    l2 = jnp.full((16,), -1e30, jnp.float32)
    for h in range(H):
        p1 = jnp.zeros((16,), jnp.float32)
        p2 = jnp.zeros((16,), jnp.float32)
        for q in range(4):
            j = h * 4 + q
            bq = xr_buf[pl.ds(dl * HC + j * 16, 16)]
            aj = att_buf[pl.ds(j * 16, 16)]
            a1 = row1[pl.ds(j * 16, 16)]
            a2 = row2[pl.ds(j * 16, 16)]
            z1 = a1 + bq
            z1 = jnp.maximum(z1, 0.2 * z1)
            z2 = a2 + bq
            z2 = jnp.maximum(z2, 0.2 * z2)
            p1 = p1 + z1 * aj
            p2 = p2 + z2 * aj
        l1 = jnp.where(iota == h, _lanesum(p1), l1)
        l2 = jnp.where(iota == h, _lanesum(p2), l2)
    mrow = mden[pl.ds(dl * 32, 16)]
    mnew = jnp.maximum(mrow, jnp.maximum(l1, l2))
    r8 = jnp.exp(mrow - mnew)
    e1 = jnp.exp(l1 - mnew)
    e2 = jnp.exp(l2 - mnew)
    mden[pl.ds(dl * 32, 16)] = mnew
    mden[pl.ds(dl * 32 + 16, 16)] = (
        mden[pl.ds(dl * 32 + 16, 16)] * r8 + e1 + e2)
    for h in range(H):
        sh = r8[h]
        eh1 = e1[h]
        eh2 = e2[h]
        for q in range(4):
            j = h * 4 + q
            o = dl * HC + j * 16
            acc[pl.ds(o, 16)] = (acc[pl.ds(o, 16)] * sh
                                 + eh1 * row1[pl.ds(j * 16, 16)]
                                 + eh2 * row2[pl.ds(j * 16, 16)])


def _edge_work(u, dl, grows, xr_buf, att_buf, mden, acc):
    iota = lax.iota(jnp.int32, 16)
    row = grows.at[u]
    lvec = jnp.full((16,), -1e30, jnp.float32)
    for h in range(H):
        ph = jnp.zeros((16,), jnp.float32)
        for q in range(4):
            j = h * 4 + q
            a = row[pl.ds(j * 16, 16)]
            bq = xr_buf[pl.ds(dl * HC + j * 16, 16)]
            z = a + bq
            z = jnp.maximum(z, 0.2 * z)
            ph = ph + z * att_buf[pl.ds(j * 16, 16)]
        lvec = jnp.where(iota == h, _lanesum(ph), lvec)
    mrow = mden[pl.ds(dl * 32, 16)]
    mnew = jnp.maximum(mrow, lvec)
    r8 = jnp.exp(mrow - mnew)
    e8 = jnp.exp(lvec - mnew)
    mden[pl.ds(dl * 32, 16)] = mnew
    mden[pl.ds(dl * 32 + 16, 16)] = mden[pl.ds(dl * 32 + 16, 16)] * r8 + e8
    for h in range(H):
        sh = r8[h]
        eh = e8[h]
        for q in range(4):
            j = h * 4 + q
            o = dl * HC + j * 16
            acc[pl.ds(o, 16)] = acc[pl.ds(o, 16)] * sh + eh * row[pl.ds(j * 16, 16)]


@functools.lru_cache(maxsize=1)
def _sc_edge_kernel():
    return functools.partial(
        pl.kernel, mesh=_mesh(),
        out_type=jax.ShapeDtypeStruct((NP * HC,), jnp.float32),
        scratch_types=[
            pltpu.VMEM((SEG,), jnp.int32),
            pltpu.VMEM((CH * HC,), jnp.float32),
            pltpu.VMEM((CH * HC,), jnp.float32),
            pltpu.VMEM((CH * 32,), jnp.float32),
            pltpu.VMEM((2, 16, HC), jnp.float32),
            pltpu.VMEM((512,), jnp.float32),
            pltpu.VMEM((NOFF,), jnp.int32),
            pltpu.SemaphoreType.DMA((2,)),
        ],
    )(_sc_edge_body)


def _sc_edge_body(xl_hbm, xr_hbm, pk_hbm, noff_hbm, att_hbm, out_hbm,
                  ebuf, xr_buf, acc, mden, grows2, att_buf, noff_buf, sems):
    w = lax.axis_index("s") * 2 + lax.axis_index("c")
    pltpu.sync_copy(noff_hbm, noff_buf)
    pltpu.sync_copy(att_hbm, att_buf)
    iota = lax.iota(jnp.int32, 16)
    zf = jnp.zeros((16,), jnp.float32)
    neg = jnp.full((16,), -1e30, jnp.float32)

    def chunk_body(k, carry):
        c = w + k * NW

        @pl.when(c < NCH)
        def _():
            base = c * CH
            bo = pl.multiple_of(base * HC, 16384)
            pltpu.sync_copy(xr_hbm.at[pl.ds(bo, CH * HC)],
                            xr_buf.at[pl.ds(0, CH * HC)])

            def zacc(j, cc):
                acc[pl.ds(j * 16, 16)] = zf
                return cc

            lax.fori_loop(0, CH * HC // 16, zacc, 0)

            def ztab(j, cc):
                mden[pl.ds(j * 32, 16)] = neg
                mden[pl.ds(j * 32 + 16, 16)] = zf
                return cc

            lax.fori_loop(0, CH, ztab, 0)

            start_c = noff_buf[pl.ds(base, 16)][0]
            end_c = noff_buf[pl.ds(base + CH, 16)][0]
            start_al = start_c & -8
            nseg = (end_c - start_al + SEG - 1) // SEG

            def seg_body(sg, cc):
                seg_base = pl.multiple_of(start_al + sg * SEG, 8)
                pltpu.sync_copy(pk_hbm.at[pl.ds(seg_base, SEG)], ebuf)
                ng = jnp.minimum(SEG // 16, (end_c - seg_base + 15) // 16)

                def srcg_of(g):
                    pkv = ebuf[pl.ds(g * 16, 16)]
                    pos = seg_base + g * 16 + iota
                    val = jnp.logical_and(pos >= start_c, pos < end_c)
                    return pkv, val, jnp.where(
                        val, lax.bitwise_and(pkv, 16383), 0)

                def issue(g):
                    _, _, srcg = srcg_of(g)
                    pltpu.async_copy(xl_hbm.at[srcg], grows2.at[g & 1],
                                     sems.at[g & 1])

                issue(0)

                def grp_body(g, cc2):
                    @pl.when(g + 1 < ng)
                    def _():
                        issue(g + 1)
                    pkv, val, srcg = srcg_of(g)
                    pltpu.make_async_copy(
                        xl_hbm.at[srcg], grows2.at[g & 1],
                        sems.at[g & 1]).wait()
                    pos0 = seg_base + g * 16
                    grows = grows2.at[g & 1]
                    dlv = lax.shift_right_logical(pkv, 14) - base
                    dl_lo = _treemin(jnp.where(val, dlv, CH - 1))[0]
                    dl_hi = _treemax(jnp.where(val, dlv, 0))[0]

                    def run_body(dl, cc3):
                        ovn = noff_buf[pl.ds(base + dl, 16)]
                        ulo = jnp.maximum(ovn[0] - pos0, 0)
                        uhi = jnp.minimum(ovn[1] - pos0, 16)
                        nn = jnp.maximum(uhi - ulo, 0)

                        def quad_body(t, cc4):
                            _edge_work4(ulo + t * 4, dl, grows, xr_buf,
                                        att_buf, mden, acc)
                            return cc4

                        lax.fori_loop(0, nn // 4, quad_body, 0)
                        rem = nn % 4

                        @pl.when(rem >= 2)
                        def _():
                            def pair_tail(t, cc4):
                                _edge_work2(uhi - rem, dl, grows, xr_buf,
                                            att_buf, mden, acc)
                                return cc4

                            lax.fori_loop(0, 1, pair_tail, 0)

                        @pl.when(rem % 2 == 1)
                        def _():
                            def tail_body(u, cc4):
                                _edge_work(u, dl, grows, xr_buf, att_buf,
                                           mden, acc)
                                return cc4

                            lax.fori_loop(uhi - 1, uhi, tail_body, 0)

                        return cc3

                    lax.fori_loop(dl_lo, dl_hi + 1, run_body, 0)
                    return cc2

                lax.fori_loop(0, ng, grp_body, 0)
                return cc

            lax.fori_loop(0, nseg, seg_body, 0)

            def fin_body(dl, cc):
                dv = mden[pl.ds(dl * 32 + 16, 16)]
                inv = 1.0 / (dv + 1e-16)
                for h in range(H):
                    ih = inv[h]
                    for q in range(4):
                        o = dl * HC + (h * 4 + q) * 16
                        acc[pl.ds(o, 16)] = acc[pl.ds(o, 16)] * ih
                return cc

            lax.fori_loop(0, CH, fin_body, 0)
            pltpu.sync_copy(acc.at[pl.ds(0, CH * HC)],
                            out_hbm.at[pl.ds(bo, CH * HC)])

        return carry

    lax.fori_loop(0, (NCH + NW - 1) // NW, chunk_body, 0)


def _pad_rows(a, rows):
    return jnp.pad(a, ((0, rows - a.shape[0]),) + ((0, 0),) * (a.ndim - 1))


def kernel(x, edge, batch, y_type, Wl0, Wr0, att0, b0, g0, be0,
           Wl1, Wr1, att1, b1, g1, be1, Wl2, Wr2, att2, b2, g2, be2,
           Wl3, Wr3, att3, b3, g3, be3, emb, Wfam, bfam, Wtyp, btyp):
    Wls = [Wl0, Wl1, Wl2, Wl3]
    Wrs = [Wr0, Wr1, Wr2, Wr3]
    atts = [att0, att1, att2, att3]
    bs = [b0, b1, b2, b3]
    gs = [g0, g1, g2, g3]
    bes = [be0, be1, be2, be3]

    loop = jnp.arange(N, dtype=jnp.int32)
    src = jnp.concatenate([edge[0].astype(jnp.int32), loop])
    dst = jnp.concatenate([edge[1].astype(jnp.int32), loop])
    pk = jnp.sort(dst * 16384 + src)
    noff = jnp.searchsorted(pk, jnp.arange(NOFF, dtype=jnp.int32) * 16384,
                            side='left').astype(jnp.int32)
    pk = jnp.pad(pk, (0, EPAD - ET), constant_values=16383 * 16384)

    xp = _pad_rows(x, NP)
    batp = jnp.pad(batch.astype(jnp.int32), (0, NP - N),
                   constant_values=B).reshape(NP, 1)
    oh = (y_type[:, None] == jnp.arange(B)[None, :]).astype(jnp.float32)
    embp = jnp.pad(emb, ((0, B - NT), (0, 128 - TE)))
    wf4 = jnp.pad(Wfam[3 * HC:], ((0, 128 - TE), (0, 0)))
    wfamp = jnp.pad(jnp.concatenate([Wfam[:3 * HC], wf4], axis=0),
                    ((0, 0), (0, 128 - NF)))
    wt4 = jnp.pad(Wtyp[3 * HC:], ((0, 128 - TE), (0, 0)))
    wtypp = jnp.pad(jnp.concatenate([Wtyp[:3 * HC], wt4], axis=0),
                    ((0, 0), (0, 128 - NT)))
    bfamp = jnp.pad(bfam, (0, 128 - NF)).reshape(1, 128)
    btypp = jnp.pad(btyp, (0, 128 - NT)).reshape(1, 128)

    h = xp
    xl, xr = _proj(xp)(xp, Wls[0], Wrs[0])
    for i in range(L):
        gat_flat = _sc_edge_kernel()(xl, xr.reshape(-1), pk, noff,
                                     atts[i].reshape(-1))
        gat = gat_flat.reshape(NP, HC)
        b2 = bs[i].reshape(1, HC)
        g2 = gs[i].reshape(1, HC)
        be2 = bes[i].reshape(1, HC)
        if i < L - 1:
            h, xl, xr = _mid(gat, h, b2, g2, be2, Wls[i + 1], Wrs[i + 1],
                             SKIP[i])
        else:
            _, _, _, o1, o2 = _final(gat, h, b2, g2, be2, batp, oh, embp,
                                     wfamp, bfamp, wtypp, btypp)
    return (o1[:, :NF], o2[:, :NT])
